# Initial kernel scaffold; baseline (speedup 1.0000x reference)
import jax
import jax.numpy as jnp
from jax.experimental import pallas as pl


def kernel(snip_feature, params):
    raise NotImplementedError("write your pallas kernel here")


# speedup vs baseline: 13.9767x; 13.9767x over previous
import jax
import jax.numpy as jnp
from jax.experimental import pallas as pl
from jax.experimental.pallas import tpu as pltpu
from jax.experimental.pallas import tpu_sc as plsc

F32 = jnp.float32
FEAT = 128
K = 6
GCN_G = 32
CONV_G = 4
BATCH = 4
SEQ = 2048
WIDTH = 128
KTILE = 256
STILE = 256
GW = 128

_HI = jax.lax.Precision.HIGHEST


def _mm(a, b):
    return jax.lax.dot_general(a, b, (((1,), (0,)), ((), ())),
                               precision=_HI, preferred_element_type=F32)


def _mm_nt(a, b):
    return jax.lax.dot_general(a, b, (((1,), (1,)), ((), ())),
                               precision=_HI, preferred_element_type=F32)


def _shift_down(x):
    return jnp.concatenate([jnp.zeros((1, x.shape[1]), x.dtype), x[:-1]], axis=0)


def _shift_up(x):
    return jnp.concatenate([x[1:], jnp.zeros((1, x.shape[1]), x.dtype)], axis=0)


def _dense_taps(w, groups):
    o, ig, kw = w.shape
    og = o // groups
    i_full = ig * groups
    wd = jnp.zeros((o, i_full, kw), w.dtype)
    for g in range(groups):
        wd = wd.at[g * og:(g + 1) * og, g * ig:(g + 1) * ig, :].set(
            w[g * og:(g + 1) * og])
    return [wd[:, :, d].T for d in range(kw)]


def _backbone(xT, w0, w1, w2, bias):
    def body(x_ref, w0_ref, w1_ref, w2_ref, b_ref, o_ref):
        x = x_ref[0]
        acc = _mm(_shift_down(x), w0_ref[...])
        acc = acc + _mm(x, w1_ref[...])
        acc = acc + _mm(_shift_up(x), w2_ref[...])
        o_ref[0] = jnp.maximum(acc + b_ref[...], 0.0)

    return pl.pallas_call(
        body,
        grid=(BATCH,),
        in_specs=[
            pl.BlockSpec((1, SEQ, FEAT), lambda b: (b, 0, 0)),
            pl.BlockSpec((FEAT, FEAT), lambda b: (0, 0)),
            pl.BlockSpec((FEAT, FEAT), lambda b: (0, 0)),
            pl.BlockSpec((FEAT, FEAT), lambda b: (0, 0)),
            pl.BlockSpec((1, FEAT), lambda b: (0, 0)),
        ],
        out_specs=pl.BlockSpec((1, SEQ, FEAT), lambda b: (b, 0, 0)),
        out_shape=jax.ShapeDtypeStruct((BATCH, SEQ, FEAT), F32),
    )(xT, w0, w1, w2, bias)


def _temporal(xT, w1T, t2a, t2b, t2c, w3T, wnT, wxT, b1, b2, b3, bs1):
    def body(x_ref, w1_ref, a_ref, bb_ref, c_ref, w3_ref, wn_ref, wx_ref,
             b1_ref, b2_ref, b3_ref, bs1_ref, tout_ref, yn_ref, yx_ref):
        x = x_ref[0]
        t1 = jnp.maximum(_mm(x, w1_ref[...]) + b1_ref[...], 0.0)
        t2 = jnp.maximum(
            _mm(_shift_down(t1), a_ref[...]) + _mm(t1, bb_ref[...])
            + _mm(_shift_up(t1), c_ref[...]) + b2_ref[...], 0.0)
        tout_ref[0] = _mm(t2, w3_ref[...]) + b3_ref[...]
        yn_ref[0] = _mm(x, wn_ref[...])
        yx_ref[0] = _mm(x, wx_ref[...]) + bs1_ref[...]

    full = pl.BlockSpec((1, SEQ, FEAT), lambda b: (b, 0, 0))
    wspec = pl.BlockSpec((FEAT, FEAT), lambda b: (0, 0))
    bspec = pl.BlockSpec((1, FEAT), lambda b: (0, 0))
    return pl.pallas_call(
        body,
        grid=(BATCH,),
        in_specs=[full, wspec, wspec, wspec, wspec, wspec, wspec, wspec,
                  bspec, bspec, bspec, bspec],
        out_specs=[full, full, full],
        out_shape=[jax.ShapeDtypeStruct((BATCH, SEQ, FEAT), F32)] * 3,
    )(xT, w1T, t2a, t2b, t2c, w3T, wnT, wxT, b1, b2, b3, bs1)


def _knn(xT):
    def body(xf_ref, xt_ref, o_ref):
        b = pl.program_id(0)
        xf = xf_ref[0]
        xt = xt_ref[0]
        xx_row = _mm_nt(jnp.ones((1, FEAT), F32), xf * xf)
        xx_col = jnp.sum(xt * xt, axis=1, keepdims=True)
        ip = _mm_nt(xt, xf)
        pd = ((-xx_row) + 2.0 * ip) - xx_col
        iota = jax.lax.broadcasted_iota(jnp.int32, (KTILE, SEQ), 1)
        cols = []
        neg = jnp.float32(-jnp.inf)
        for _ in range(K):
            m = jnp.max(pd, axis=1, keepdims=True)
            am = jnp.min(jnp.where(pd >= m, iota, SEQ), axis=1, keepdims=True)
            cols.append(am + b * SEQ)
            pd = jnp.where(iota == am, neg, pd)
        pad = jnp.zeros((KTILE, 1), jnp.int32)
        o_ref[0] = jnp.concatenate(cols + [pad, pad], axis=1)

    return pl.pallas_call(
        body,
        grid=(BATCH, SEQ // KTILE),
        in_specs=[
            pl.BlockSpec((1, SEQ, FEAT), lambda b, i: (b, 0, 0)),
            pl.BlockSpec((1, KTILE, FEAT), lambda b, i: (b, i, 0)),
        ],
        out_specs=pl.BlockSpec((1, KTILE, 8), lambda b, i: (b, i, 0)),
        out_shape=jax.ShapeDtypeStruct((BATCH, SEQ, 8), jnp.int32),
    )(xT, xT)


def _sc_gather(data, inds):
    m = inds.shape[1]
    mesh = plsc.VectorSubcoreMesh(core_axis_name="c", subcore_axis_name="s")

    @pl.kernel(out_type=jax.ShapeDtypeStruct((m, data.shape[1]), data.dtype),
               mesh=mesh)
    def gather_kernel(x_hbm, i_hbm, o_hbm):
        def body(i_vmem, o_vmem):
            pltpu.sync_copy(x_hbm.at[i_vmem.at[0]], o_vmem)

        pltpu.emit_pipeline(
            body,
            grid=(m // GW,),
            in_specs=[pl.BlockSpec((1, GW), index_map=lambda i: (0, i))],
            out_specs=[pl.BlockSpec((GW, data.shape[1]),
                                    index_map=lambda i: (i, 0))],
            core_axis_name=("c", "s"),
            dimension_semantics=(pltpu.PARALLEL,),
        )(i_hbm, o_hbm)

    return gather_kernel(data, inds)


def _sbranch(gath, yx, tout, xres, orig, w2T, w3T, b2, b3):
    nrows = BATCH * SEQ
    with_orig = orig is not None

    def body(*refs):
        if with_orig:
            (g_ref, yx_ref, tout_ref, xr_ref, or_ref,
             w2_ref, w3_ref, b2_ref, b3_ref, o_ref) = refs
        else:
            (g_ref, yx_ref, tout_ref, xr_ref,
             w2_ref, w3_ref, b2_ref, b3_ref, o_ref) = refs
        yxv = yx_ref[...]
        m = None
        for k in range(K):
            s1 = jnp.maximum(g_ref[k] + yxv, 0.0)
            s2 = jnp.maximum(_mm(s1, w2_ref[...]) + b2_ref[...], 0.0)
            s3 = _mm(s2, w3_ref[...])
            m = s3 if m is None else jnp.maximum(m, s3)
        h = jnp.maximum((tout_ref[...] + xr_ref[...]) + (m + b3_ref[...]), 0.0)
        if with_orig:
            h = h + or_ref[...]
        o_ref[...] = h

    row = pl.BlockSpec((STILE, FEAT), lambda i: (i, 0))
    ins = [gath, yx, tout, xres] + ([orig] if with_orig else []) + [w2T, w3T, b2, b3]
    in_specs = (
        [pl.BlockSpec((K, STILE, FEAT), lambda i: (0, i, 0)), row, row, row]
        + ([row] if with_orig else [])
        + [pl.BlockSpec((WIDTH, WIDTH), lambda i: (0, 0)),
           pl.BlockSpec((WIDTH, FEAT), lambda i: (0, 0)),
           pl.BlockSpec((1, WIDTH), lambda i: (0, 0)),
           pl.BlockSpec((1, FEAT), lambda i: (0, 0))])
    return pl.pallas_call(
        body,
        grid=(nrows // STILE,),
        in_specs=in_specs,
        out_specs=row,
        out_shape=jax.ShapeDtypeStruct((nrows, FEAT), F32),
    )(*ins)


def _gcnext(xT, p, g, origT):
    w1T = p[g + '_t1_w'][:, :, 0].T
    b1 = p[g + '_t1_b'][None, :]
    t2a, t2b, t2c = _dense_taps(p[g + '_t2_w'], GCN_G)
    b2 = p[g + '_t2_b'][None, :]
    w3T = p[g + '_t3_w'][:, :, 0].T
    b3 = p[g + '_t3_b'][None, :]
    s1w = p[g + '_s1_w'][:, :, 0, 0]
    wnT = s1w[:, :FEAT].T
    wxT = s1w[:, FEAT:].T
    bs1 = p[g + '_s1_b'][None, :]
    s2T = _dense_taps(p[g + '_s2_w'][:, :, :, 0], GCN_G)[0]
    bs2 = p[g + '_s2_b'][None, :]
    s3T = p[g + '_s3_w'][:, :, 0, 0].T
    bs3 = p[g + '_s3_b'][None, :]

    tout, yn, yx = _temporal(xT, w1T, t2a, t2b, t2c, w3T, wnT, wxT,
                             b1, b2, b3, bs1)
    idxg = _knn(xT)
    inds = jnp.transpose(idxg[:, :, :K], (2, 0, 1)).reshape(1, K * BATCH * SEQ)
    gath = _sc_gather(yn.reshape(BATCH * SEQ, FEAT), inds)
    out = _sbranch(gath.reshape(K, BATCH * SEQ, FEAT),
                   yx.reshape(BATCH * SEQ, FEAT),
                   tout.reshape(BATCH * SEQ, FEAT),
                   xT.reshape(BATCH * SEQ, FEAT),
                   None if origT is None else origT.reshape(BATCH * SEQ, FEAT),
                   s2T, s3T, bs2, bs3)
    return out.reshape(BATCH, SEQ, FEAT)


def kernel(snip_feature, params):
    p = params
    x0T = jnp.transpose(snip_feature, (0, 2, 1))

    s = p['bn_g'] / jnp.sqrt(jnp.float32(1.0 + 1e-05))
    taps = _dense_taps(p['bb_w'], CONV_G)
    taps = [t * s[None, :] for t in taps]
    bb_bias = (p['bb_b'] * s + p['bn_b'])[None, :]
    h = _backbone(x0T, taps[0], taps[1], taps[2], bb_bias)

    h = _gcnext(h, p, 'g1', None)
    h = _gcnext(h, p, 'g2', x0T)
    return jnp.transpose(h, (0, 2, 1))


# speedup vs baseline: 21.1019x; 1.5098x over previous
import jax
import jax.numpy as jnp
from jax.experimental import pallas as pl
from jax.experimental.pallas import tpu as pltpu
from jax.experimental.pallas import tpu_sc as plsc

F32 = jnp.float32
FEAT = 128
K = 6
GCN_G = 32
CONV_G = 4
BATCH = 4
SEQ = 2048
WIDTH = 128
KTILE = 256
STILE = 256
GW = 128

_HI = jax.lax.Precision.DEFAULT


def _mm(a, b):
    return jax.lax.dot_general(a, b, (((1,), (0,)), ((), ())),
                               precision=_HI, preferred_element_type=F32)


def _mm_nt(a, b):
    return jax.lax.dot_general(a, b, (((1,), (1,)), ((), ())),
                               precision=_HI, preferred_element_type=F32)


def _shift_down(x):
    return jnp.concatenate([jnp.zeros((1, x.shape[1]), x.dtype), x[:-1]], axis=0)


def _shift_up(x):
    return jnp.concatenate([x[1:], jnp.zeros((1, x.shape[1]), x.dtype)], axis=0)


def _dense_taps(w, groups):
    o, ig, kw = w.shape
    og = o // groups
    i_full = ig * groups
    wd = jnp.zeros((o, i_full, kw), w.dtype)
    for g in range(groups):
        wd = wd.at[g * og:(g + 1) * og, g * ig:(g + 1) * ig, :].set(
            w[g * og:(g + 1) * og])
    return [wd[:, :, d].T for d in range(kw)]


def _conv3(x, wcat):
    u = jnp.concatenate([_shift_down(x), x, _shift_up(x)], axis=1)
    return _mm(u, wcat)


def _backbone(xT, wcat, bias):
    def body(x_ref, w_ref, b_ref, o_ref):
        o_ref[0] = jnp.maximum(_conv3(x_ref[0], w_ref[...]) + b_ref[...], 0.0)

    return pl.pallas_call(
        body,
        grid=(BATCH,),
        in_specs=[
            pl.BlockSpec((1, SEQ, FEAT), lambda b: (b, 0, 0)),
            pl.BlockSpec((3 * FEAT, FEAT), lambda b: (0, 0)),
            pl.BlockSpec((1, FEAT), lambda b: (0, 0)),
        ],
        out_specs=pl.BlockSpec((1, SEQ, FEAT), lambda b: (b, 0, 0)),
        out_shape=jax.ShapeDtypeStruct((BATCH, SEQ, FEAT), F32),
    )(xT, wcat, bias)


def _temporal(xT, wcat1, w2cat, w3T, b1, b2, b3, bs1):
    def body(x_ref, wc_ref, w2_ref, w3_ref,
             b1_ref, b2_ref, b3_ref, bs1_ref, tout_ref, yn_ref, yx_ref):
        x = x_ref[0]
        y = _mm(x, wc_ref[...])
        t1 = jnp.maximum(y[:, :FEAT] + b1_ref[...], 0.0)
        yn_ref[0] = y[:, FEAT:2 * FEAT]
        yx_ref[0] = y[:, 2 * FEAT:] + bs1_ref[...]
        t2 = jnp.maximum(_conv3(t1, w2_ref[...]) + b2_ref[...], 0.0)
        tout_ref[0] = _mm(t2, w3_ref[...]) + b3_ref[...]

    full = pl.BlockSpec((1, SEQ, FEAT), lambda b: (b, 0, 0))
    bspec = pl.BlockSpec((1, FEAT), lambda b: (0, 0))
    return pl.pallas_call(
        body,
        grid=(BATCH,),
        in_specs=[full,
                  pl.BlockSpec((FEAT, 3 * FEAT), lambda b: (0, 0)),
                  pl.BlockSpec((3 * FEAT, FEAT), lambda b: (0, 0)),
                  pl.BlockSpec((FEAT, FEAT), lambda b: (0, 0)),
                  bspec, bspec, bspec, bspec],
        out_specs=[full, full, full],
        out_shape=[jax.ShapeDtypeStruct((BATCH, SEQ, FEAT), F32)] * 3,
    )(xT, wcat1, w2cat, w3T, b1, b2, b3, bs1)


def _knn(xT):
    def body(xf_ref, xt_ref, o_ref, xx_ref):
        b = pl.program_id(0)
        xf = xf_ref[0]
        xt = xt_ref[0]

        @pl.when(pl.program_id(1) == 0)
        def _():
            xx_ref[...] = _mm_nt(jnp.ones((1, FEAT), F32), xf * xf)

        xx_row = xx_ref[...]
        xx_col = jnp.sum(xt * xt, axis=1, keepdims=True)
        ip = _mm_nt(xt, xf)
        pd = ((-xx_row) + 2.0 * ip) - xx_col
        iota = jax.lax.broadcasted_iota(jnp.int32, (KTILE, SEQ), 1)
        cols = []
        neg = jnp.float32(-jnp.inf)
        for _ in range(K):
            m = jnp.max(pd, axis=1, keepdims=True)
            am = jnp.min(jnp.where(pd >= m, iota, SEQ), axis=1, keepdims=True)
            cols.append(am + b * SEQ)
            pd = jnp.where(iota == am, neg, pd)
        pad = jnp.zeros((KTILE, 1), jnp.int32)
        o_ref[0] = jnp.concatenate(cols + [pad, pad], axis=1)

    return pl.pallas_call(
        body,
        grid=(BATCH, SEQ // KTILE),
        in_specs=[
            pl.BlockSpec((1, SEQ, FEAT), lambda b, i: (b, 0, 0)),
            pl.BlockSpec((1, KTILE, FEAT), lambda b, i: (b, i, 0)),
        ],
        out_specs=pl.BlockSpec((1, KTILE, 8), lambda b, i: (b, i, 0)),
        out_shape=jax.ShapeDtypeStruct((BATCH, SEQ, 8), jnp.int32),
        scratch_shapes=[pltpu.VMEM((1, SEQ), F32)],
    )(xT, xT)


def _sc_gather(data, inds):
    m = inds.shape[1]
    mesh = plsc.VectorSubcoreMesh(core_axis_name="c", subcore_axis_name="s")

    @pl.kernel(out_type=jax.ShapeDtypeStruct((m, data.shape[1]), data.dtype),
               mesh=mesh)
    def gather_kernel(x_hbm, i_hbm, o_hbm):
        def body(i_vmem, o_vmem):
            pltpu.sync_copy(x_hbm.at[i_vmem.at[0]], o_vmem)

        pltpu.emit_pipeline(
            body,
            grid=(m // GW,),
            in_specs=[pl.BlockSpec((1, GW), index_map=lambda i: (0, i))],
            out_specs=[pl.BlockSpec((GW, data.shape[1]),
                                    index_map=lambda i: (i, 0))],
            core_axis_name=("c", "s"),
            dimension_semantics=(pltpu.PARALLEL,),
        )(i_hbm, o_hbm)

    return gather_kernel(data, inds)


def _sbranch(gath, yx, tout, xres, orig, w2T, w3T, b2, b3):
    nrows = BATCH * SEQ
    with_orig = orig is not None

    def body(*refs):
        if with_orig:
            (g_ref, yx_ref, tout_ref, xr_ref, or_ref,
             w2_ref, w3_ref, b2_ref, b3_ref, o_ref) = refs
        else:
            (g_ref, yx_ref, tout_ref, xr_ref,
             w2_ref, w3_ref, b2_ref, b3_ref, o_ref) = refs
        yxv = yx_ref[...]
        s1 = jnp.concatenate(
            [jnp.maximum(g_ref[k] + yxv, 0.0) for k in range(K)], axis=0)
        s2 = jnp.maximum(_mm(s1, w2_ref[...]) + b2_ref[...], 0.0)
        s3 = _mm(s2, w3_ref[...])
        m = s3[:STILE]
        for k in range(1, K):
            m = jnp.maximum(m, s3[k * STILE:(k + 1) * STILE])
        h = jnp.maximum((tout_ref[...] + xr_ref[...]) + (m + b3_ref[...]), 0.0)
        if with_orig:
            h = h + or_ref[...]
        o_ref[...] = h

    row = pl.BlockSpec((STILE, FEAT), lambda i: (i, 0))
    ins = [gath, yx, tout, xres] + ([orig] if with_orig else []) + [w2T, w3T, b2, b3]
    in_specs = (
        [pl.BlockSpec((K, STILE, FEAT), lambda i: (0, i, 0)), row, row, row]
        + ([row] if with_orig else [])
        + [pl.BlockSpec((WIDTH, WIDTH), lambda i: (0, 0)),
           pl.BlockSpec((WIDTH, FEAT), lambda i: (0, 0)),
           pl.BlockSpec((1, WIDTH), lambda i: (0, 0)),
           pl.BlockSpec((1, FEAT), lambda i: (0, 0))])
    return pl.pallas_call(
        body,
        grid=(nrows // STILE,),
        in_specs=in_specs,
        out_specs=row,
        out_shape=jax.ShapeDtypeStruct((nrows, FEAT), F32),
    )(*ins)


def _gcnext(xT, p, g, origT):
    w1T = p[g + '_t1_w'][:, :, 0].T
    b1 = p[g + '_t1_b'][None, :]
    w2cat = jnp.concatenate(_dense_taps(p[g + '_t2_w'], GCN_G), axis=0)
    b2 = p[g + '_t2_b'][None, :]
    w3T = p[g + '_t3_w'][:, :, 0].T
    b3 = p[g + '_t3_b'][None, :]
    s1w = p[g + '_s1_w'][:, :, 0, 0]
    wnT = s1w[:, :FEAT].T
    wxT = s1w[:, FEAT:].T
    bs1 = p[g + '_s1_b'][None, :]
    s2T = _dense_taps(p[g + '_s2_w'][:, :, :, 0], GCN_G)[0]
    bs2 = p[g + '_s2_b'][None, :]
    s3T = p[g + '_s3_w'][:, :, 0, 0].T
    bs3 = p[g + '_s3_b'][None, :]

    wcat1 = jnp.concatenate([w1T, wnT, wxT], axis=1)
    tout, yn, yx = _temporal(xT, wcat1, w2cat, w3T, b1, b2, b3, bs1)
    idxg = _knn(xT)
    inds = jnp.transpose(idxg[:, :, :K], (2, 0, 1)).reshape(1, K * BATCH * SEQ)
    gath = _sc_gather(yn.reshape(BATCH * SEQ, FEAT), inds)
    out = _sbranch(gath.reshape(K, BATCH * SEQ, FEAT),
                   yx.reshape(BATCH * SEQ, FEAT),
                   tout.reshape(BATCH * SEQ, FEAT),
                   xT.reshape(BATCH * SEQ, FEAT),
                   None if origT is None else origT.reshape(BATCH * SEQ, FEAT),
                   s2T, s3T, bs2, bs3)
    return out.reshape(BATCH, SEQ, FEAT)


def kernel(snip_feature, params):
    p = params
    x0T = jnp.transpose(snip_feature, (0, 2, 1))

    s = p['bn_g'] / jnp.sqrt(jnp.float32(1.0 + 1e-05))
    taps = _dense_taps(p['bb_w'], CONV_G)
    wcat = jnp.concatenate([t * s[None, :] for t in taps], axis=0)
    bb_bias = (p['bb_b'] * s + p['bn_b'])[None, :]
    h = _backbone(x0T, wcat, bb_bias)

    h = _gcnext(h, p, 'g1', None)
    h = _gcnext(h, p, 'g2', x0T)
    return jnp.transpose(h, (0, 2, 1))


# speedup vs baseline: 26.4549x; 1.2537x over previous
import jax
import jax.numpy as jnp
from jax.experimental import pallas as pl
from jax.experimental.pallas import tpu as pltpu
from jax.experimental.pallas import tpu_sc as plsc

F32 = jnp.float32
FEAT = 128
K = 6
GCN_G = 32
CONV_G = 4
BATCH = 4
SEQ = 2048
WIDTH = 128
KTILE = 512
STILE = 256
GW = 256

_HI = jax.lax.Precision.DEFAULT


def _mm(a, b):
    return jax.lax.dot_general(a, b, (((1,), (0,)), ((), ())),
                               precision=_HI, preferred_element_type=F32)


def _mm_nt(a, b):
    return jax.lax.dot_general(a, b, (((1,), (1,)), ((), ())),
                               precision=_HI, preferred_element_type=F32)


def _shift_down(x):
    return jnp.concatenate([jnp.zeros((1, x.shape[1]), x.dtype), x[:-1]], axis=0)


def _shift_up(x):
    return jnp.concatenate([x[1:], jnp.zeros((1, x.shape[1]), x.dtype)], axis=0)


def _dense_taps(w, groups):
    o, ig, kw = w.shape
    og = o // groups
    i_full = ig * groups
    oi = jax.lax.broadcasted_iota(jnp.int32, (o, i_full), 0)
    ii = jax.lax.broadcasted_iota(jnp.int32, (o, i_full), 1)
    mask = (oi // og) == (ii // ig)
    wt = jnp.tile(w, (1, groups, 1))
    wd = jnp.where(mask[:, :, None], wt, 0.0)
    return [wd[:, :, d].T for d in range(kw)]


def _conv3(x, wcat):
    u = jnp.concatenate([_shift_down(x), x, _shift_up(x)], axis=1)
    return _mm(u, wcat)


def _backbone(xT, wcat, bias):
    def body(x_ref, w_ref, b_ref, o_ref):
        o_ref[0] = jnp.maximum(_conv3(x_ref[0], w_ref[...]) + b_ref[...], 0.0)

    return pl.pallas_call(
        body,
        grid=(BATCH,),
        in_specs=[
            pl.BlockSpec((1, SEQ, FEAT), lambda b: (b, 0, 0)),
            pl.BlockSpec((3 * FEAT, FEAT), lambda b: (0, 0)),
            pl.BlockSpec((1, FEAT), lambda b: (0, 0)),
        ],
        out_specs=pl.BlockSpec((1, SEQ, FEAT), lambda b: (b, 0, 0)),
        out_shape=jax.ShapeDtypeStruct((BATCH, SEQ, FEAT), F32),
    )(xT, wcat, bias)


def _temporal(xT, wcat1, w2cat, w3T, b1, b2, b3, bs1):
    def body(x_ref, wc_ref, w2_ref, w3_ref,
             b1_ref, b2_ref, b3_ref, bs1_ref, tout_ref, yn_ref, yx_ref):
        x = x_ref[0]
        y = _mm(x, wc_ref[...])
        t1 = jnp.maximum(y[:, :FEAT] + b1_ref[...], 0.0)
        yn_ref[0] = y[:, FEAT:2 * FEAT]
        yx_ref[0] = y[:, 2 * FEAT:] + bs1_ref[...]
        t2 = jnp.maximum(_conv3(t1, w2_ref[...]) + b2_ref[...], 0.0)
        tout_ref[0] = _mm(t2, w3_ref[...]) + b3_ref[...]

    full = pl.BlockSpec((1, SEQ, FEAT), lambda b: (b, 0, 0))
    bspec = pl.BlockSpec((1, FEAT), lambda b: (0, 0))
    return pl.pallas_call(
        body,
        grid=(BATCH,),
        in_specs=[full,
                  pl.BlockSpec((FEAT, 3 * FEAT), lambda b: (0, 0)),
                  pl.BlockSpec((3 * FEAT, FEAT), lambda b: (0, 0)),
                  pl.BlockSpec((FEAT, FEAT), lambda b: (0, 0)),
                  bspec, bspec, bspec, bspec],
        out_specs=[full, full, full],
        out_shape=[jax.ShapeDtypeStruct((BATCH, SEQ, FEAT), F32)] * 3,
    )(xT, wcat1, w2cat, w3T, b1, b2, b3, bs1)


def _knn(xT):
    def body(xf_ref, xt_ref, o_ref, xx_ref):
        b = pl.program_id(0)
        xf = xf_ref[0]
        xt = xt_ref[0]

        @pl.when(pl.program_id(1) == 0)
        def _():
            xx_ref[...] = _mm_nt(jnp.ones((1, FEAT), F32), xf * xf)

        xx_row = xx_ref[...]
        xx_col = jnp.sum(xt * xt, axis=1, keepdims=True)
        ip2 = _mm_nt(xt + xt, xf)
        pd = ((-xx_row) + ip2) - xx_col
        nchunk = SEQ // 128
        iota128 = jax.lax.broadcasted_iota(jnp.int32, (KTILE, 128), 1)
        cols = []
        neg = jnp.float32(-jnp.inf)
        for _ in range(K):
            m = jnp.max(pd, axis=1, keepdims=True)
            cm = jnp.full((KTILE, 128), nchunk, jnp.int32)
            for c in reversed(range(nchunk)):
                cm = jnp.where(pd[:, c * 128:(c + 1) * 128] >= m, c, cm)
            col = jnp.where(cm < nchunk, cm * 128 + iota128, SEQ)
            am = jnp.min(col, axis=1, keepdims=True)
            cols.append(am + b * SEQ)
            parts = []
            for c in range(nchunk):
                cond = iota128 == (am - c * 128)
                parts.append(jnp.where(cond, neg, pd[:, c * 128:(c + 1) * 128]))
            pd = jnp.concatenate(parts, axis=1)
        pad = jnp.zeros((KTILE, 1), jnp.int32)
        o_ref[0] = jnp.concatenate(cols + [pad, pad], axis=1)

    return pl.pallas_call(
        body,
        grid=(BATCH, SEQ // KTILE),
        in_specs=[
            pl.BlockSpec((1, SEQ, FEAT), lambda b, i: (b, 0, 0)),
            pl.BlockSpec((1, KTILE, FEAT), lambda b, i: (b, i, 0)),
        ],
        out_specs=pl.BlockSpec((1, KTILE, 8), lambda b, i: (b, i, 0)),
        out_shape=jax.ShapeDtypeStruct((BATCH, SEQ, 8), jnp.int32),
        scratch_shapes=[pltpu.VMEM((1, SEQ), F32)],
    )(xT, xT)


def _sc_gather(data, inds):
    m = inds.shape[1]
    mesh = plsc.VectorSubcoreMesh(core_axis_name="c", subcore_axis_name="s")

    @pl.kernel(out_type=jax.ShapeDtypeStruct((m, data.shape[1]), data.dtype),
               mesh=mesh)
    def gather_kernel(x_hbm, i_hbm, o_hbm):
        def body(i_vmem, o_vmem):
            pltpu.sync_copy(x_hbm.at[i_vmem.at[0]], o_vmem)

        pltpu.emit_pipeline(
            body,
            grid=(m // GW,),
            in_specs=[pl.BlockSpec((1, GW), index_map=lambda i: (0, i))],
            out_specs=[pl.BlockSpec((GW, data.shape[1]),
                                    index_map=lambda i: (i, 0))],
            core_axis_name=("c", "s"),
            dimension_semantics=(pltpu.PARALLEL,),
        )(i_hbm, o_hbm)

    return gather_kernel(data, inds)


def _sbranch(gath, yx, tout, xres, orig, w2T, w3T, b2, b3):
    nrows = BATCH * SEQ
    with_orig = orig is not None

    def body(*refs):
        if with_orig:
            (g_ref, yx_ref, tout_ref, xr_ref, or_ref,
             w2_ref, w3_ref, b2_ref, b3_ref, o_ref) = refs
        else:
            (g_ref, yx_ref, tout_ref, xr_ref,
             w2_ref, w3_ref, b2_ref, b3_ref, o_ref) = refs
        yxv = yx_ref[...]
        s1 = jnp.concatenate(
            [jnp.maximum(g_ref[k] + yxv, 0.0) for k in range(K)], axis=0)
        s2 = jnp.maximum(_mm(s1, w2_ref[...]) + b2_ref[...], 0.0)
        s3 = _mm(s2, w3_ref[...])
        m = s3[:STILE]
        for k in range(1, K):
            m = jnp.maximum(m, s3[k * STILE:(k + 1) * STILE])
        h = jnp.maximum((tout_ref[...] + xr_ref[...]) + (m + b3_ref[...]), 0.0)
        if with_orig:
            h = h + or_ref[...]
        o_ref[...] = h

    row = pl.BlockSpec((STILE, FEAT), lambda i: (i, 0))
    ins = [gath, yx, tout, xres] + ([orig] if with_orig else []) + [w2T, w3T, b2, b3]
    in_specs = (
        [pl.BlockSpec((K, STILE, FEAT), lambda i: (0, i, 0)), row, row, row]
        + ([row] if with_orig else [])
        + [pl.BlockSpec((WIDTH, WIDTH), lambda i: (0, 0)),
           pl.BlockSpec((WIDTH, FEAT), lambda i: (0, 0)),
           pl.BlockSpec((1, WIDTH), lambda i: (0, 0)),
           pl.BlockSpec((1, FEAT), lambda i: (0, 0))])
    return pl.pallas_call(
        body,
        grid=(nrows // STILE,),
        in_specs=in_specs,
        out_specs=row,
        out_shape=jax.ShapeDtypeStruct((nrows, FEAT), F32),
    )(*ins)


def _gcnext(xT, p, g, origT):
    w1T = p[g + '_t1_w'][:, :, 0].T
    b1 = p[g + '_t1_b'][None, :]
    w2cat = jnp.concatenate(_dense_taps(p[g + '_t2_w'], GCN_G), axis=0)
    b2 = p[g + '_t2_b'][None, :]
    w3T = p[g + '_t3_w'][:, :, 0].T
    b3 = p[g + '_t3_b'][None, :]
    s1w = p[g + '_s1_w'][:, :, 0, 0]
    wnT = s1w[:, :FEAT].T
    wxT = s1w[:, FEAT:].T
    bs1 = p[g + '_s1_b'][None, :]
    s2T = _dense_taps(p[g + '_s2_w'][:, :, :, 0], GCN_G)[0]
    bs2 = p[g + '_s2_b'][None, :]
    s3T = p[g + '_s3_w'][:, :, 0, 0].T
    bs3 = p[g + '_s3_b'][None, :]

    wcat1 = jnp.concatenate([w1T, wnT, wxT], axis=1)
    tout, yn, yx = _temporal(xT, wcat1, w2cat, w3T, b1, b2, b3, bs1)
    idxg = _knn(xT)
    inds = jnp.transpose(idxg[:, :, :K], (2, 0, 1)).reshape(1, K * BATCH * SEQ)
    gath = _sc_gather(yn.reshape(BATCH * SEQ, FEAT), inds)
    out = _sbranch(gath.reshape(K, BATCH * SEQ, FEAT),
                   yx.reshape(BATCH * SEQ, FEAT),
                   tout.reshape(BATCH * SEQ, FEAT),
                   xT.reshape(BATCH * SEQ, FEAT),
                   None if origT is None else origT.reshape(BATCH * SEQ, FEAT),
                   s2T, s3T, bs2, bs3)
    return out.reshape(BATCH, SEQ, FEAT)


def kernel(snip_feature, params):
    p = params
    x0T = jnp.transpose(snip_feature, (0, 2, 1))

    s = p['bn_g'] / jnp.sqrt(jnp.float32(1.0 + 1e-05))
    taps = _dense_taps(p['bb_w'], CONV_G)
    wcat = jnp.concatenate([t * s[None, :] for t in taps], axis=0)
    bb_bias = (p['bb_b'] * s + p['bn_b'])[None, :]
    h = _backbone(x0T, wcat, bb_bias)

    h = _gcnext(h, p, 'g1', None)
    h = _gcnext(h, p, 'g2', x0T)
    return jnp.transpose(h, (0, 2, 1))
